# baseline (speedup 1.0000x reference)
import functools

import jax
import jax.numpy as jnp
from jax import lax
from jax.experimental import pallas as pl
from jax.experimental.pallas import tpu as pltpu
from jax.experimental.pallas import tpu_sc as plsc

B, E, C = 4096, 16, 1000
_NC, _NS, _L = 2, 16, 16
NW = _NC * _NS
SPW = B // NW
ELEMS = SPW * E


def _sc_gather_body(table_hbm, tgt_hbm, out_hbm, tgt_v, idx_v, out_v, sem, osem):
    wid = lax.axis_index("s") * _NC + lax.axis_index("c")
    pltpu.sync_copy(tgt_hbm.at[pl.ds(wid * SPW, SPW)], tgt_v)
    iota = lax.iota(jnp.int32, _L)
    base = wid * 1024 + iota
    tparts = []
    for i in range(SPW // _L):
        t16 = tgt_v[pl.ds(i * _L, _L)]
        tparts.append((t16 >> 3) * 32768 + (t16 & 7) * 128 + (base + i * _L))
    copies = []
    for e in range(E):
        for i in range(SPW // _L):
            idx_v[pl.ds(e * 128 + i * _L, _L)] = tparts[i] + e * (B * C)
        if e % 8 == 7:
            h = e // 8
            copies.append(
                pltpu.async_copy(table_hbm.at[idx_v.at[pl.ds(h * 1024, 1024)]],
                                 out_v.at[pl.ds(h * 1024, 1024)], sem))
    for cp in copies:
        cp.wait()
    o1 = pltpu.async_copy(out_v.at[pl.ds(0, 1024)],
                          out_hbm.at[pl.ds(wid * 1024, 1024)], osem)
    o2 = pltpu.async_copy(out_v.at[pl.ds(1024, 1024)],
                          out_hbm.at[pl.ds(B * 8 + wid * 1024, 1024)], osem)
    o1.wait()
    o2.wait()


_sc_gather = functools.partial(
    pl.kernel,
    mesh=plsc.VectorSubcoreMesh(core_axis_name="c", subcore_axis_name="s"),
    out_type=jax.ShapeDtypeStruct((B * E,), jnp.float32),
    scratch_types=[
        pltpu.VMEM((SPW,), jnp.int32),
        pltpu.VMEM((ELEMS,), jnp.int32),
        pltpu.VMEM((ELEMS,), jnp.float32),
        pltpu.SemaphoreType.DMA,
        pltpu.SemaphoreType.DMA,
    ],
    compiler_params=pltpu.CompilerParams(needs_layout_passes=False),
)(_sc_gather_body)


def _tc_loss_body(g_ref, gate_ref, o_ref):
    e_logp = jnp.log(g_ref[...] + 1e-15)
    o_ref[0, 0] = jnp.sum(jnp.abs(gate_ref[...] - e_logp)) * (1.0 / B)


def kernel(outputs, expert_outputs, gate_outputs, targets):
    table = (expert_outputs.transpose(1, 2, 0)
             .reshape(E, C // 8, 8, B // 128, 128)
             .transpose(0, 1, 3, 2, 4)
             .reshape(B * E * C))
    gathered = _sc_gather(table, targets.astype(jnp.int32))
    gate_phys = (gate_outputs.T
                 .reshape(2, 8, B // 128, 128)
                 .transpose(0, 2, 1, 3)
                 .reshape(B * E // 128, 128))
    loss = pl.pallas_call(
        _tc_loss_body,
        out_shape=jax.ShapeDtypeStruct((1, 1), jnp.float32),
        out_specs=pl.BlockSpec(memory_space=pltpu.SMEM),
    )(gathered.reshape(B * E // 128, 128), gate_phys)
    return loss[0, 0]


# speedup vs baseline: 1.0177x; 1.0177x over previous
import functools

import jax
import jax.numpy as jnp
from jax import lax
from jax.experimental import pallas as pl
from jax.experimental.pallas import tpu as pltpu
from jax.experimental.pallas import tpu_sc as plsc

B, E, C = 4096, 16, 1000
_NC, _NS, _L = 2, 16, 16
NW = _NC * _NS
SPW = B // NW
ELEMS = SPW * E


def _sc_gather_body(table_hbm, tgt_hbm, out_hbm, tgt_v, idx_v, out_v,
                    sem0, sem1, osem):
    wid = lax.axis_index("s") * _NC + lax.axis_index("c")
    pltpu.sync_copy(tgt_hbm.at[pl.ds(wid * SPW, SPW)], tgt_v)
    iota = lax.iota(jnp.int32, _L)
    base = wid * 1024 + iota
    tparts = []
    for i in range(SPW // _L):
        t16 = tgt_v[pl.ds(i * _L, _L)]
        tparts.append((t16 >> 3) * 32768 + (t16 & 7) * 128 + (base + i * _L))
    copies = []
    for e in range(E):
        for i in range(SPW // _L):
            idx_v[pl.ds(e * 128 + i * _L, _L)] = tparts[i] + e * (B * C)
        if e % 8 == 7:
            h = e // 8
            copies.append(
                pltpu.async_copy(table_hbm.at[idx_v.at[pl.ds(h * 1024, 1024)]],
                                 out_v.at[pl.ds(h * 1024, 1024)],
                                 sem0 if h == 0 else sem1))
    copies[0].wait()
    o1 = pltpu.async_copy(out_v.at[pl.ds(0, 1024)],
                          out_hbm.at[pl.ds(wid * 1024, 1024)], osem)
    copies[1].wait()
    o2 = pltpu.async_copy(out_v.at[pl.ds(1024, 1024)],
                          out_hbm.at[pl.ds(B * 8 + wid * 1024, 1024)], osem)
    o1.wait()
    o2.wait()


_sc_gather = functools.partial(
    pl.kernel,
    mesh=plsc.VectorSubcoreMesh(core_axis_name="c", subcore_axis_name="s"),
    out_type=jax.ShapeDtypeStruct((B * E,), jnp.float32),
    scratch_types=[
        pltpu.VMEM((SPW,), jnp.int32),
        pltpu.VMEM((ELEMS,), jnp.int32),
        pltpu.VMEM((ELEMS,), jnp.float32),
        pltpu.SemaphoreType.DMA,
        pltpu.SemaphoreType.DMA,
        pltpu.SemaphoreType.DMA,
    ],
    compiler_params=pltpu.CompilerParams(needs_layout_passes=False),
)(_sc_gather_body)


def _tc_loss_body(g_ref, gate_ref, o_ref):
    e_logp = jnp.log(g_ref[...] + 1e-15)
    o_ref[0, 0] = jnp.sum(jnp.abs(gate_ref[...] - e_logp)) * (1.0 / B)


def kernel(outputs, expert_outputs, gate_outputs, targets):
    table = (expert_outputs.transpose(1, 2, 0)
             .reshape(E, C // 8, 8, B // 128, 128)
             .transpose(0, 1, 3, 2, 4)
             .reshape(B * E * C))
    gathered = _sc_gather(table, targets.astype(jnp.int32))
    gate_phys = (gate_outputs.T
                 .reshape(2, 8, B // 128, 128)
                 .transpose(0, 2, 1, 3)
                 .reshape(B * E // 128, 128))
    loss = pl.pallas_call(
        _tc_loss_body,
        out_shape=jax.ShapeDtypeStruct((1, 1), jnp.float32),
        out_specs=pl.BlockSpec(memory_space=pltpu.SMEM),
    )(gathered.reshape(B * E // 128, 128), gate_phys)
    return loss[0, 0]
